# initial kernel scaffold (unmeasured)
import jax
import jax.numpy as jnp
from jax import lax
from jax.experimental import pallas as pl
from jax.experimental.pallas import tpu as pltpu

T = 512
D = 1024
V_SHARD = 8192


def kernel(x, W):
    xb = x.astype(jnp.bfloat16)
    Wb = W.astype(jnp.bfloat16)

    def body(x_ref, w_ref, out_ref, mine_ref, theirs_ref, send_sem, recv_sem):
        my_x = lax.axis_index("x")
        my_y = lax.axis_index("y")
        my_z = lax.axis_index("z")
        nbr = (1 - my_x, my_y, my_z)

        barrier = pltpu.get_barrier_semaphore()
        pl.semaphore_signal(
            barrier, inc=1, device_id=nbr, device_id_type=pl.DeviceIdType.MESH
        )
        pl.semaphore_wait(barrier, 1)

        mine_ref[...] = jnp.dot(
            x_ref[...], w_ref[...], preferred_element_type=jnp.float32
        ).astype(jnp.bfloat16)

        rdma = pltpu.make_async_remote_copy(
            src_ref=mine_ref,
            dst_ref=theirs_ref,
            send_sem=send_sem,
            recv_sem=recv_sem,
            device_id=nbr,
            device_id_type=pl.DeviceIdType.MESH,
        )
        rdma.start()
        rdma.wait()

        mine = mine_ref[...].astype(jnp.float32)
        theirs = theirs_ref[...].astype(jnp.float32)
        m = jnp.maximum(
            mine.max(axis=-1, keepdims=True), theirs.max(axis=-1, keepdims=True)
        )
        e0 = jnp.exp(mine - m)
        e1 = jnp.exp(theirs - m)
        s = e0.sum(axis=-1, keepdims=True) + e1.sum(axis=-1, keepdims=True)
        off_mine = my_x * V_SHARD
        off_theirs = (1 - my_x) * V_SHARD
        out_ref[:, pl.ds(off_mine, V_SHARD)] = e0 / s
        out_ref[:, pl.ds(off_theirs, V_SHARD)] = e1 / s

    return pl.pallas_call(
        body,
        out_shape=jax.ShapeDtypeStruct((T, 2 * V_SHARD), jnp.float32),
        in_specs=[
            pl.BlockSpec(memory_space=pltpu.VMEM),
            pl.BlockSpec(memory_space=pltpu.VMEM),
        ],
        out_specs=pl.BlockSpec(memory_space=pltpu.VMEM),
        scratch_shapes=[
            pltpu.VMEM((T, V_SHARD), jnp.bfloat16),
            pltpu.VMEM((T, V_SHARD), jnp.bfloat16),
            pltpu.SemaphoreType.DMA,
            pltpu.SemaphoreType.DMA,
        ],
        compiler_params=pltpu.CompilerParams(collective_id=0),
    )(xb, Wb)


# baseline (device time: 256328 ns/iter reference)
import jax
import jax.numpy as jnp
from jax import lax
from jax.experimental import pallas as pl
from jax.experimental.pallas import tpu as pltpu

T = 512
D = 1024
V_SHARD = 8192
V = 2 * V_SHARD
CHUNK = 1024
NC = V_SHARD // CHUNK


def kernel(x, W):
    xb = x.astype(jnp.bfloat16)
    Wb = W.astype(jnp.bfloat16)

    def body(x_ref, w_ref, out_ref, send_sem, recv_sem):
        my_x = lax.axis_index("x")
        my_y = lax.axis_index("y")
        my_z = lax.axis_index("z")
        nbr = (1 - my_x, my_y, my_z)

        barrier = pltpu.get_barrier_semaphore()
        pl.semaphore_signal(
            barrier, inc=1, device_id=nbr, device_id_type=pl.DeviceIdType.MESH
        )
        pl.semaphore_wait(barrier, 1)

        for c in range(NC):
            out_ref[c] = jnp.dot(
                x_ref[...],
                w_ref[:, pl.ds(c * CHUNK, CHUNK)],
                preferred_element_type=jnp.float32,
            ).astype(jnp.bfloat16)

        rdma = pltpu.make_async_remote_copy(
            src_ref=out_ref.at[pl.ds(0, NC)],
            dst_ref=out_ref.at[pl.ds(NC, NC)],
            send_sem=send_sem,
            recv_sem=recv_sem,
            device_id=nbr,
            device_id_type=pl.DeviceIdType.MESH,
        )
        rdma.start()
        rdma.wait()

        def max_step(k, m):
            return jnp.maximum(
                m, out_ref[k].astype(jnp.float32).max(axis=-1, keepdims=True)
            )

        m = lax.fori_loop(
            0, 2 * NC, max_step, jnp.full((T, 1), -jnp.inf, jnp.float32)
        )

        def exp_step(k, s):
            e = jnp.exp(out_ref[k].astype(jnp.float32) - m)
            out_ref[k] = e.astype(jnp.bfloat16)
            return s + e.sum(axis=-1, keepdims=True)

        s = lax.fori_loop(0, 2 * NC, exp_step, jnp.zeros((T, 1), jnp.float32))
        r = 1.0 / s

        def norm_step(k, carry):
            out_ref[k] = (out_ref[k].astype(jnp.float32) * r).astype(
                jnp.bfloat16
            )
            return carry

        lax.fori_loop(0, 2 * NC, norm_step, 0)

    out3 = pl.pallas_call(
        body,
        out_shape=jax.ShapeDtypeStruct((2 * NC, T, CHUNK), jnp.bfloat16),
        in_specs=[
            pl.BlockSpec(memory_space=pltpu.VMEM),
            pl.BlockSpec(memory_space=pltpu.VMEM),
        ],
        out_specs=pl.BlockSpec(memory_space=pltpu.VMEM),
        scratch_shapes=[
            pltpu.SemaphoreType.DMA,
            pltpu.SemaphoreType.DMA,
        ],
        compiler_params=pltpu.CompilerParams(
            collective_id=0, vmem_limit_bytes=63 * 1024 * 1024
        ),
    )(xb, Wb)

    flat = out3.transpose(1, 0, 2).reshape(T, V)
    my_x = lax.axis_index("x")
    return jnp.roll(flat, my_x * V_SHARD, axis=1)


# device time: 120359 ns/iter; 2.1297x vs baseline; 2.1297x over previous
import jax
import jax.numpy as jnp
from jax import lax
from jax.experimental import pallas as pl
from jax.experimental.pallas import tpu as pltpu

T = 512
D = 1024
V_SHARD = 8192
V = 2 * V_SHARD
CHUNK = 1024
NC = V_SHARD // CHUNK


def kernel(x, W):
    def body(
        x_ref,
        w_ref,
        out_ref,
        logits_ref,
        emine_ref,
        wbuf_ref,
        w_sems,
        send_sems,
        recv_sems,
        out_sems,
    ):
        my_x = lax.axis_index("x")
        my_y = lax.axis_index("y")
        my_z = lax.axis_index("z")
        nbr = (1 - my_x, my_y, my_z)

        barrier = pltpu.get_barrier_semaphore()
        pl.semaphore_signal(
            barrier, inc=1, device_id=nbr, device_id_type=pl.DeviceIdType.MESH
        )
        pl.semaphore_wait(barrier, 1)

        def w_dma(c):
            return pltpu.make_async_copy(
                w_ref.at[:, pl.ds(c * CHUNK, CHUNK)],
                wbuf_ref.at[c % 2],
                w_sems.at[c % 2],
            )

        def chunk_rdma(c):
            return pltpu.make_async_remote_copy(
                src_ref=logits_ref.at[c],
                dst_ref=logits_ref.at[NC + c],
                send_sem=send_sems.at[c],
                recv_sem=recv_sems.at[c],
                device_id=nbr,
                device_id_type=pl.DeviceIdType.MESH,
            )

        xb = x_ref[...].astype(jnp.bfloat16)

        w_dma(0).start()
        for c in range(NC):
            if c + 1 < NC:
                w_dma(c + 1).start()
            w_dma(c).wait()
            logits_ref[c] = jnp.dot(
                xb,
                wbuf_ref[c % 2].astype(jnp.bfloat16),
                preferred_element_type=jnp.float32,
            ).astype(jnp.bfloat16)
            chunk_rdma(c).start()

        lane = lax.broadcasted_iota(jnp.int32, (T, 2 * NC), 1)

        def chunk_stats(k, carry, e_dst):
            m_all, s_all = carry
            l = logits_ref[k].astype(jnp.float32)
            m_k = l.max(axis=-1, keepdims=True)
            e = jnp.exp(l - m_k)
            e_dst[...] = e.astype(jnp.bfloat16)
            s_k = e.sum(axis=-1, keepdims=True)
            sel = lane == k
            return jnp.where(sel, m_k, m_all), jnp.where(sel, s_k, s_all)

        stats0 = (
            jnp.full((T, 2 * NC), -jnp.inf, jnp.float32),
            jnp.zeros((T, 2 * NC), jnp.float32),
        )

        def mine_step(k, carry):
            return chunk_stats(k, carry, emine_ref.at[k])

        carry = lax.fori_loop(0, NC, mine_step, stats0)

        def theirs_step(k, carry):
            pltpu.make_async_remote_copy(
                src_ref=logits_ref.at[k - NC],
                dst_ref=logits_ref.at[k],
                send_sem=send_sems.at[k - NC],
                recv_sem=recv_sems.at[k - NC],
                device_id=nbr,
                device_id_type=pl.DeviceIdType.MESH,
            ).wait_recv()
            return chunk_stats(k, carry, logits_ref.at[k])

        m_all, s_all = lax.fori_loop(NC, 2 * NC, theirs_step, carry)

        m = m_all.max(axis=-1, keepdims=True)
        s = (s_all * jnp.exp(m_all - m)).sum(axis=-1, keepdims=True)
        g_all = jnp.exp(m_all - m) / s

        for c in range(NC):
            chunk_rdma(c).wait_send()

        def scale_step(k, carry):
            sel = (lane == k).astype(jnp.float32)
            g = (g_all * sel).sum(axis=-1, keepdims=True)

            @pl.when(k < NC)
            def _():
                logits_ref[k] = (
                    emine_ref[k].astype(jnp.float32) * g
                ).astype(jnp.bfloat16)

            @pl.when(k >= NC)
            def _():
                logits_ref[k] = (
                    logits_ref[k].astype(jnp.float32) * g
                ).astype(jnp.bfloat16)

            col = jnp.where(
                my_x == 0, k, lax.rem(k + NC, 2 * NC)
            ) * CHUNK
            pltpu.make_async_copy(
                logits_ref.at[k],
                out_ref.at[:, pl.ds(col, CHUNK)],
                out_sems.at[k],
            ).start()
            return carry

        lax.fori_loop(0, 2 * NC, scale_step, 0)

        for j in range(2 * NC):
            col = jnp.where(
                my_x == 0, j * CHUNK, ((j + NC) % (2 * NC)) * CHUNK
            )
            pltpu.make_async_copy(
                logits_ref.at[j],
                out_ref.at[:, pl.ds(col, CHUNK)],
                out_sems.at[j],
            ).wait()

    return pl.pallas_call(
        body,
        out_shape=jax.ShapeDtypeStruct((T, V), jnp.bfloat16),
        in_specs=[
            pl.BlockSpec(memory_space=pltpu.VMEM),
            pl.BlockSpec(memory_space=pltpu.MemorySpace.HBM),
        ],
        out_specs=pl.BlockSpec(memory_space=pltpu.MemorySpace.HBM),
        scratch_shapes=[
            pltpu.VMEM((2 * NC, T, CHUNK), jnp.bfloat16),
            pltpu.VMEM((NC, T, CHUNK), jnp.bfloat16),
            pltpu.VMEM((2, D, CHUNK), jnp.float32),
            pltpu.SemaphoreType.DMA((2,)),
            pltpu.SemaphoreType.DMA((NC,)),
            pltpu.SemaphoreType.DMA((NC,)),
            pltpu.SemaphoreType.DMA((2 * NC,)),
        ],
        compiler_params=pltpu.CompilerParams(
            collective_id=0, vmem_limit_bytes=63 * 1024 * 1024
        ),
    )(x, W)


# device time: 91995 ns/iter; 2.7863x vs baseline; 1.3083x over previous
import jax
import jax.numpy as jnp
from jax import lax
from jax.experimental import pallas as pl
from jax.experimental.pallas import tpu as pltpu

T = 512
D = 1024
V_SHARD = 8192
V = 2 * V_SHARD
CHUNK = 1024
NC = V_SHARD // CHUNK


def kernel(x, W):
    def body(
        x_ref,
        w_ref,
        out_ref,
        logits_ref,
        emine_ref,
        wbuf_ref,
        w_sems,
        sx_sems,
        rx_sems,
        sy_sems,
        ry_sems,
        sz_sems,
        rz_sems,
        out_sems,
    ):
        my_x = lax.axis_index("x")
        my_y = lax.axis_index("y")
        my_z = lax.axis_index("z")
        nbr_x = (1 - my_x, my_y, my_z)
        nbr_y = (my_x, 1 - my_y, my_z)
        nbr_z = (my_x, my_y, 1 - my_z)

        q_x = 2 * my_y + my_z
        q_y = 2 * (1 - my_y) + my_z

        def perm(j):
            return lax.rem(2 * q_x + j, NC)

        barrier = pltpu.get_barrier_semaphore()
        for nbr in (nbr_x, nbr_y, nbr_z):
            pl.semaphore_signal(
                barrier,
                inc=1,
                device_id=nbr,
                device_id_type=pl.DeviceIdType.MESH,
            )
        pl.semaphore_wait(barrier, 3)

        def w_dma(j):
            return pltpu.make_async_copy(
                w_ref.at[:, pl.ds(perm(j) * CHUNK, CHUNK)],
                wbuf_ref.at[j % 2],
                w_sems.at[j % 2],
            )

        def rdma(src_slot, dst_slot, send_sem, recv_sem, nbr):
            return pltpu.make_async_remote_copy(
                src_ref=logits_ref.at[src_slot],
                dst_ref=logits_ref.at[dst_slot],
                send_sem=send_sem,
                recv_sem=recv_sem,
                device_id=nbr,
                device_id_type=pl.DeviceIdType.MESH,
            )

        xb = x_ref[...].astype(jnp.bfloat16)

        w_dma(0).start()
        for j in range(NC):
            if j + 1 < NC:
                w_dma(j + 1).start()
            w_dma(j).wait()
            logits_ref[j] = jnp.dot(
                xb,
                wbuf_ref[j % 2].astype(jnp.bfloat16),
                preferred_element_type=jnp.float32,
            ).astype(jnp.bfloat16)
            if j < 2:
                rdma(
                    j, NC + 2 * q_x + j, sx_sems.at[j], rx_sems.at[j], nbr_x
                ).start()

        for j in range(2):
            rdma(
                0, NC + 2 * q_x + j, sx_sems.at[j], rx_sems.at[j], nbr_x
            ).wait_recv()
            rdma(
                NC + 2 * q_x + j,
                NC + 2 * q_x + j,
                sy_sems.at[j],
                ry_sems.at[j],
                nbr_y,
            ).start()
            zi = 2 * my_y + j
            rdma(
                NC + 2 * q_x + j,
                NC + 2 * q_x + j,
                sz_sems.at[zi],
                rz_sems.at[zi],
                nbr_z,
            ).start()
        for j in range(2):
            rdma(
                0, NC + 2 * q_y + j, sy_sems.at[j], ry_sems.at[j], nbr_y
            ).wait_recv()
            zi = 2 * (1 - my_y) + j
            rdma(
                NC + 2 * q_y + j,
                NC + 2 * q_y + j,
                sz_sems.at[zi],
                rz_sems.at[zi],
                nbr_z,
            ).start()

        lane = lax.broadcasted_iota(jnp.int32, (T, 2 * NC), 1)

        def chunk_stats(k, carry, e_dst):
            m_all, s_all = carry
            l = logits_ref[k].astype(jnp.float32)
            m_k = l.max(axis=-1, keepdims=True)
            e = jnp.exp(l - m_k)
            if e_dst is not None:
                e_dst[...] = e.astype(jnp.bfloat16)
            s_k = e.sum(axis=-1, keepdims=True)
            sel = lane == k
            return jnp.where(sel, m_k, m_all), jnp.where(sel, s_k, s_all)

        stats0 = (
            jnp.full((T, 2 * NC), -jnp.inf, jnp.float32),
            jnp.zeros((T, 2 * NC), jnp.float32),
        )

        def mine_step(k, carry):
            return chunk_stats(k, carry, emine_ref.at[k])

        carry = lax.fori_loop(0, NC, mine_step, stats0)

        def xq_step(j, carry):
            return chunk_stats(NC + 2 * q_x + j, carry, None)

        carry = lax.fori_loop(0, 2, xq_step, carry)

        def yq_step(j, carry):
            return chunk_stats(NC + 2 * q_y + j, carry, None)

        carry = lax.fori_loop(0, 2, yq_step, carry)

        def zq_step(i, carry):
            c = 2 * (1 - my_z) + i + jnp.where(i >= 2, 2, 0)
            k = NC + c
            rdma(0, k, sz_sems.at[i], rz_sems.at[i], nbr_z).wait_recv()
            return chunk_stats(k, carry, logits_ref.at[k])

        m_all, s_all = lax.fori_loop(0, 4, zq_step, carry)

        m = m_all.max(axis=-1, keepdims=True)
        s = (s_all * jnp.exp(m_all - m)).sum(axis=-1, keepdims=True)
        g_all = jnp.exp(m_all - m) / s

        for j in range(2):
            rdma(j, NC, sx_sems.at[j], rx_sems.at[j], nbr_x).wait_send()
            rdma(j, NC, sy_sems.at[j], ry_sems.at[j], nbr_y).wait_send()
        for i in range(4):
            rdma(0, NC, sz_sems.at[i], rz_sems.at[i], nbr_z).wait_send()

        def scale_step(k, carry):
            sel = (lane == k).astype(jnp.float32)
            g = (g_all * sel).sum(axis=-1, keepdims=True)
            m_k = (m_all * sel).sum(axis=-1, keepdims=True)
            quarter = lax.div(k - NC, 2)
            is_fwd = jnp.logical_and(
                k >= NC,
                jnp.logical_or(quarter == q_x, quarter == q_y),
            )

            @pl.when(k < NC)
            def _():
                logits_ref[k] = (
                    emine_ref[lax.rem(k, NC)].astype(jnp.float32) * g
                ).astype(jnp.bfloat16)

            @pl.when(is_fwd)
            def _():
                logits_ref[k] = (
                    jnp.exp(logits_ref[k].astype(jnp.float32) - m_k) * g
                ).astype(jnp.bfloat16)

            @pl.when(jnp.logical_and(k >= NC, jnp.logical_not(is_fwd)))
            def _():
                logits_ref[k] = (
                    logits_ref[k].astype(jnp.float32) * g
                ).astype(jnp.bfloat16)

            col_block = jnp.where(
                k < NC,
                my_x * NC + perm(k),
                (1 - my_x) * NC + (k - NC),
            )
            pltpu.make_async_copy(
                logits_ref.at[k],
                out_ref.at[:, pl.ds(col_block * CHUNK, CHUNK)],
                out_sems.at[k],
            ).start()
            return carry

        lax.fori_loop(0, 2 * NC, scale_step, 0)

        for j in range(2 * NC):
            col_block = jnp.where(
                j < NC,
                my_x * NC + perm(j),
                (1 - my_x) * NC + (j - NC),
            )
            pltpu.make_async_copy(
                logits_ref.at[j],
                out_ref.at[:, pl.ds(col_block * CHUNK, CHUNK)],
                out_sems.at[j],
            ).wait()

    return pl.pallas_call(
        body,
        out_shape=jax.ShapeDtypeStruct((T, V), jnp.bfloat16),
        in_specs=[
            pl.BlockSpec(memory_space=pltpu.VMEM),
            pl.BlockSpec(memory_space=pltpu.MemorySpace.HBM),
        ],
        out_specs=pl.BlockSpec(memory_space=pltpu.MemorySpace.HBM),
        scratch_shapes=[
            pltpu.VMEM((2 * NC, T, CHUNK), jnp.bfloat16),
            pltpu.VMEM((NC, T, CHUNK), jnp.bfloat16),
            pltpu.VMEM((2, D, CHUNK), jnp.float32),
            pltpu.SemaphoreType.DMA((2,)),
            pltpu.SemaphoreType.DMA((2,)),
            pltpu.SemaphoreType.DMA((2,)),
            pltpu.SemaphoreType.DMA((2,)),
            pltpu.SemaphoreType.DMA((2,)),
            pltpu.SemaphoreType.DMA((4,)),
            pltpu.SemaphoreType.DMA((4,)),
            pltpu.SemaphoreType.DMA((2 * NC,)),
        ],
        compiler_params=pltpu.CompilerParams(
            collective_id=0, vmem_limit_bytes=63 * 1024 * 1024
        ),
    )(x, W)


# device time: 89481 ns/iter; 2.8646x vs baseline; 1.0281x over previous
import jax
import jax.numpy as jnp
from jax import lax
from jax.experimental import pallas as pl
from jax.experimental.pallas import tpu as pltpu

T = 512
D = 1024
V_SHARD = 8192
V = 2 * V_SHARD
CHUNK = 1024
NC = V_SHARD // CHUNK
NE = NC + 4


def kernel(x, W):
    def body(
        x_ref,
        w_ref,
        out_ref,
        logits_ref,
        e_ref,
        wbuf_ref,
        w_sems,
        sx_sems,
        rx_sems,
        sy_sems,
        ry_sems,
        sz_sems,
        rz_sems,
        out_sems,
    ):
        my_x = lax.axis_index("x")
        my_y = lax.axis_index("y")
        my_z = lax.axis_index("z")
        nbr_x = (1 - my_x, my_y, my_z)
        nbr_y = (my_x, 1 - my_y, my_z)
        nbr_z = (my_x, my_y, 1 - my_z)

        q_x = 2 * my_y + my_z
        q_y = 2 * (1 - my_y) + my_z

        def perm(j):
            return lax.rem(2 * q_x + j, NC)

        barrier = pltpu.get_barrier_semaphore()
        for nbr in (nbr_x, nbr_y, nbr_z):
            pl.semaphore_signal(
                barrier,
                inc=1,
                device_id=nbr,
                device_id_type=pl.DeviceIdType.MESH,
            )
        pl.semaphore_wait(barrier, 3)

        def w_dma(j):
            return pltpu.make_async_copy(
                w_ref.at[:, pl.ds(perm(j) * CHUNK, CHUNK)],
                wbuf_ref.at[j % 2],
                w_sems.at[j % 2],
            )

        def rdma(src_slot, dst_slot, send_sem, recv_sem, nbr):
            return pltpu.make_async_remote_copy(
                src_ref=logits_ref.at[src_slot],
                dst_ref=logits_ref.at[dst_slot],
                send_sem=send_sem,
                recv_sem=recv_sem,
                device_id=nbr,
                device_id_type=pl.DeviceIdType.MESH,
            )

        xb = x_ref[...].astype(jnp.bfloat16)

        w_dma(0).start()
        for j in range(NC):
            if j + 1 < NC:
                w_dma(j + 1).start()
            w_dma(j).wait()
            logits_ref[j] = jnp.dot(
                xb,
                wbuf_ref[j % 2].astype(jnp.bfloat16),
                preferred_element_type=jnp.float32,
            ).astype(jnp.bfloat16)
            if j < 2:
                rdma(
                    j, NC + 2 * q_x + j, sx_sems.at[j], rx_sems.at[j], nbr_x
                ).start()

        for j in range(2):
            rdma(
                0, NC + 2 * q_x + j, sx_sems.at[j], rx_sems.at[j], nbr_x
            ).wait_recv()
            rdma(
                NC + 2 * q_x + j,
                NC + 2 * q_x + j,
                sy_sems.at[j],
                ry_sems.at[j],
                nbr_y,
            ).start()
            zi = 2 * my_y + j
            rdma(
                NC + 2 * q_x + j,
                NC + 2 * q_x + j,
                sz_sems.at[zi],
                rz_sems.at[zi],
                nbr_z,
            ).start()
        for j in range(2):
            rdma(
                0, NC + 2 * q_y + j, sy_sems.at[j], ry_sems.at[j], nbr_y
            ).wait_recv()
            zi = 2 * (1 - my_y) + j
            rdma(
                NC + 2 * q_y + j,
                NC + 2 * q_y + j,
                sz_sems.at[zi],
                rz_sems.at[zi],
                nbr_z,
            ).start()

        def chunk_exp(k, s_run, e_dst):
            e = jnp.exp(logits_ref[k].astype(jnp.float32))
            e_dst[...] = e.astype(jnp.bfloat16)
            return s_run + e.sum(axis=-1, keepdims=True)

        def mine_step(k, s_run):
            return chunk_exp(k, s_run, e_ref.at[k])

        s_run = lax.fori_loop(
            0, NC, mine_step, jnp.zeros((T, 1), jnp.float32)
        )

        def xq_step(j, s_run):
            return chunk_exp(NC + 2 * q_x + j, s_run, e_ref.at[NC + j])

        s_run = lax.fori_loop(0, 2, xq_step, s_run)

        def yq_step(j, s_run):
            return chunk_exp(NC + 2 * q_y + j, s_run, e_ref.at[NC + 2 + j])

        s_run = lax.fori_loop(0, 2, yq_step, s_run)

        def zq_step(i, s_run):
            c = 2 * (1 - my_z) + i + jnp.where(i >= 2, 2, 0)
            k = NC + c
            rdma(0, k, sz_sems.at[i], rz_sems.at[i], nbr_z).wait_recv()
            return chunk_exp(k, s_run, logits_ref.at[k])

        s = lax.fori_loop(0, 4, zq_step, s_run)
        r = 1.0 / s

        for j in range(2):
            rdma(j, NC, sx_sems.at[j], rx_sems.at[j], nbr_x).wait_send()
            rdma(j, NC, sy_sems.at[j], ry_sems.at[j], nbr_y).wait_send()
        for i in range(4):
            rdma(0, NC, sz_sems.at[i], rz_sems.at[i], nbr_z).wait_send()

        def scale_step(k, carry):
            c = k - NC
            quarter = lax.div(c, 2)
            e_idx = jnp.where(
                k < NC,
                k,
                jnp.where(
                    quarter == q_x,
                    NC + (c - 2 * q_x),
                    NC + 2 + (c - 2 * q_y),
                ),
            )
            in_e = jnp.logical_or(
                k < NC,
                jnp.logical_or(quarter == q_x, quarter == q_y),
            )

            @pl.when(in_e)
            def _():
                logits_ref[k] = (
                    e_ref[e_idx].astype(jnp.float32) * r
                ).astype(jnp.bfloat16)

            @pl.when(jnp.logical_not(in_e))
            def _():
                logits_ref[k] = (
                    logits_ref[k].astype(jnp.float32) * r
                ).astype(jnp.bfloat16)

            col_block = jnp.where(
                k < NC,
                my_x * NC + perm(k),
                (1 - my_x) * NC + c,
            )
            pltpu.make_async_copy(
                logits_ref.at[k],
                out_ref.at[:, pl.ds(col_block * CHUNK, CHUNK)],
                out_sems.at[k],
            ).start()
            return carry

        lax.fori_loop(0, 2 * NC, scale_step, 0)

        for j in range(2 * NC):
            col_block = jnp.where(
                j < NC,
                my_x * NC + perm(j),
                (1 - my_x) * NC + (j - NC),
            )
            pltpu.make_async_copy(
                logits_ref.at[j],
                out_ref.at[:, pl.ds(col_block * CHUNK, CHUNK)],
                out_sems.at[j],
            ).wait()

    return pl.pallas_call(
        body,
        out_shape=jax.ShapeDtypeStruct((T, V), jnp.bfloat16),
        in_specs=[
            pl.BlockSpec(memory_space=pltpu.VMEM),
            pl.BlockSpec(memory_space=pltpu.MemorySpace.HBM),
        ],
        out_specs=pl.BlockSpec(memory_space=pltpu.MemorySpace.HBM),
        scratch_shapes=[
            pltpu.VMEM((2 * NC, T, CHUNK), jnp.bfloat16),
            pltpu.VMEM((NE, T, CHUNK), jnp.bfloat16),
            pltpu.VMEM((2, D, CHUNK), jnp.float32),
            pltpu.SemaphoreType.DMA((2,)),
            pltpu.SemaphoreType.DMA((2,)),
            pltpu.SemaphoreType.DMA((2,)),
            pltpu.SemaphoreType.DMA((2,)),
            pltpu.SemaphoreType.DMA((2,)),
            pltpu.SemaphoreType.DMA((4,)),
            pltpu.SemaphoreType.DMA((4,)),
            pltpu.SemaphoreType.DMA((2 * NC,)),
        ],
        compiler_params=pltpu.CompilerParams(
            collective_id=0, vmem_limit_bytes=63 * 1024 * 1024
        ),
    )(x, W)
